# baseline (speedup 1.0000x reference)
import functools

import jax
import jax.numpy as jnp
from jax import lax
from jax.experimental import pallas as pl
from jax.experimental.pallas import tpu as pltpu
from jax.experimental.pallas import tpu_sc as plsc

N, E, DIN, H, DOUT = 10000, 320000, 128, 64, 128
HALF = N // 2
HP = 2 * H

NC, NS = 2, 16
NW = NC * NS
EPT = E // NW
CHUNK = 128
EPT_P = 10240
NCHUNKS = EPT_P // CHUNK
NBUF = 4
NITER = NCHUNKS // NBUF
NPAD = 10240
ROWS_PT = NPAD // NS
ZR = 128

_PREC = jax.lax.Precision.HIGHEST


def _dot(a, b):
    return jax.lax.dot_general(a, b, (((1,), (0,)), ((), ())),
                               precision=_PREC,
                               preferred_element_type=jnp.float32)


def _sc_segment_sum(p, src3, dst3, zeros_z):
    mesh = plsc.VectorSubcoreMesh(core_axis_name="c", subcore_axis_name="s")

    @functools.partial(
        pl.kernel,
        mesh=mesh,
        compiler_params=pltpu.CompilerParams(use_tc_tiling_on_sc=False),
        out_type=jax.ShapeDtypeStruct((NC, NPAD, H), jnp.float32),
        scratch_types=[
            pltpu.VMEM((NCHUNKS, CHUNK), jnp.int32),
            pltpu.VMEM((NCHUNKS, CHUNK), jnp.int32),
            pltpu.VMEM((CHUNK, H), jnp.float32),
            pltpu.VMEM((CHUNK, H), jnp.float32),
            pltpu.VMEM((CHUNK, H), jnp.float32),
            pltpu.VMEM((CHUNK, H), jnp.float32),
            pltpu.VMEM((ZR, H), jnp.float32),
            pltpu.VMEM_SHARED((NPAD, H), jnp.float32),
            pltpu.SemaphoreType.DMA,
            pltpu.SemaphoreType.DMA,
            pltpu.SemaphoreType.DMA,
            pltpu.SemaphoreType.DMA,
            pltpu.SemaphoreType.DMA,
            pltpu.SemaphoreType.DMA,
            pltpu.SemaphoreType.DMA,
            pltpu.SemaphoreType.DMA,
        ],
    )
    def k(p_hbm, src_hbm, dst_hbm, z_hbm, out_hbm, sbuf, dbuf,
          r0, r1, r2, r3, zbuf, acc, g0, g1, g2, g3, s0, s1, s2, s3):
        rows = [r0, r1, r2, r3]
        gsem = [g0, g1, g2, g3]
        ssem = [s0, s1, s2, s3]
        c = lax.axis_index("c")
        s = lax.axis_index("s")
        w = c * NS + s

        pltpu.sync_copy(src_hbm.at[w], sbuf)
        pltpu.sync_copy(dst_hbm.at[w], dbuf)
        for b in range(NBUF):
            pltpu.async_copy(p_hbm.at[sbuf.at[b]], rows[b], gsem[b])

        pltpu.sync_copy(z_hbm, zbuf)
        for j in range(ROWS_PT // ZR):
            pltpu.sync_copy(zbuf, acc.at[pl.ds(s * ROWS_PT + j * ZR, ZR)])
        plsc.subcore_barrier()

        def body(i, _):
            for b in range(NBUF):
                j = i * NBUF + b
                pltpu.make_async_copy(
                    p_hbm.at[sbuf.at[j]], rows[b], gsem[b]).wait()
                pltpu.async_copy(rows[b], acc.at[dbuf.at[j]], ssem[b],
                                 add=True)
                pb = (b - 1) % NBUF

                if b == 0:
                    @pl.when(i > 0)
                    def _():
                        pltpu.make_async_copy(
                            rows[pb], acc.at[dbuf.at[j - 1]], ssem[pb]).wait()
                        pltpu.async_copy(
                            p_hbm.at[sbuf.at[j - 1 + NBUF]], rows[pb],
                            gsem[pb])
                else:
                    pltpu.make_async_copy(
                        rows[pb], acc.at[dbuf.at[j - 1]], ssem[pb]).wait()

                    @pl.when(i < NITER - 1)
                    def _():
                        pltpu.async_copy(
                            p_hbm.at[sbuf.at[j - 1 + NBUF]], rows[pb],
                            gsem[pb])
            return 0

        lax.fori_loop(0, NITER, body, 0)
        pltpu.make_async_copy(rows[NBUF - 1], acc.at[dbuf.at[NCHUNKS - 1]],
                              ssem[NBUF - 1]).wait()
        plsc.subcore_barrier()

        for j in range(ROWS_PT // ZR):
            r0_ = s * ROWS_PT + j * ZR
            pltpu.async_copy(acc.at[pl.ds(r0_, ZR)],
                             out_hbm.at[c].at[pl.ds(r0_, ZR)], gsem[j % NBUF])
        for j in range(ROWS_PT // ZR):
            r0_ = s * ROWS_PT + j * ZR
            pltpu.make_async_copy(acc.at[pl.ds(r0_, ZR)],
                                  out_hbm.at[c].at[pl.ds(r0_, ZR)],
                                  gsem[j % NBUF]).wait()

    return k(p, src3, dst3, zeros_z)


_BR = 1000


def _tc_project0(x, W1d):
    def body(xt_ref, xb_ref, w_ref, o_ref):
        xp = jnp.concatenate([xt_ref[...], xb_ref[...]], axis=1)
        o_ref[...] = _dot(xp, w_ref[...])

    return pl.pallas_call(
        body,
        grid=(HALF // _BR,),
        in_specs=[
            pl.BlockSpec((_BR, DIN), lambda i: (i, 0)),
            pl.BlockSpec((_BR, DIN), lambda i: (i + HALF // _BR, 0)),
            pl.BlockSpec((2 * DIN, HP), lambda i: (0, 0)),
        ],
        out_specs=pl.BlockSpec((_BR, HP), lambda i: (i, 0)),
        out_shape=jax.ShapeDtypeStruct((HALF, HP), jnp.float32),
    )(x, x, W1d)


def _tc_block(p, acc2p, b1d, W2d, b2d, Mavg, gd, bed, W1nd):
    def body(p_ref, a_ref, b1_ref, w2_ref, b2_ref, m_ref, g_ref, be_ref,
             w1n_ref, o_ref):
        t = p_ref[...] + a_ref[0] + a_ref[1] + b1_ref[...]
        t = jnp.maximum(t, 0.0)
        u = _dot(t, w2_ref[...]) + b2_ref[...]
        mu = _dot(u, m_ref[...])
        d = u - mu
        var = _dot(d * d, m_ref[...])
        v = d / jnp.sqrt(var + 1e-5) * g_ref[...] + be_ref[...]
        v = jnp.maximum(v, 0.0)
        o_ref[...] = _dot(v, w1n_ref[...])

    return pl.pallas_call(
        body,
        grid=(HALF // _BR,),
        in_specs=[
            pl.BlockSpec((_BR, HP), lambda i: (i, 0)),
            pl.BlockSpec((NC, _BR, HP), lambda i: (0, i, 0)),
            pl.BlockSpec((1, HP), lambda i: (0, 0)),
            pl.BlockSpec((HP, HP), lambda i: (0, 0)),
            pl.BlockSpec((1, HP), lambda i: (0, 0)),
            pl.BlockSpec((HP, HP), lambda i: (0, 0)),
            pl.BlockSpec((1, HP), lambda i: (0, 0)),
            pl.BlockSpec((1, HP), lambda i: (0, 0)),
            pl.BlockSpec((HP, HP), lambda i: (0, 0)),
        ],
        out_specs=pl.BlockSpec((_BR, HP), lambda i: (i, 0)),
        out_shape=jax.ShapeDtypeStruct((HALF, HP), jnp.float32),
    )(p, acc2p, b1d, W2d, b2d, Mavg, gd, bed, W1nd)


def _tc_final(p, acc2p, b1d, W2_3d, b2_3d):
    def body(p_ref, a_ref, b1_ref, w2_ref, b2_ref, o_ref):
        t = p_ref[...] + a_ref[0] + a_ref[1] + b1_ref[...]
        t = jnp.maximum(t, 0.0)
        u = _dot(t, w2_ref[...]) + b2_ref[...]
        o_ref[0] = u[:, :DOUT]
        o_ref[1] = u[:, DOUT:]

    return pl.pallas_call(
        body,
        grid=(HALF // _BR,),
        in_specs=[
            pl.BlockSpec((_BR, HP), lambda i: (i, 0)),
            pl.BlockSpec((NC, _BR, HP), lambda i: (0, i, 0)),
            pl.BlockSpec((1, HP), lambda i: (0, 0)),
            pl.BlockSpec((HP, 2 * DOUT), lambda i: (0, 0)),
            pl.BlockSpec((1, 2 * DOUT), lambda i: (0, 0)),
        ],
        out_specs=pl.BlockSpec((2, _BR, DOUT), lambda i: (0, i, 0)),
        out_shape=jax.ShapeDtypeStruct((2, HALF, DOUT), jnp.float32),
    )(p, acc2p, b1d, W2_3d, b2_3d)


def _bd(A):
    return jnp.kron(jnp.eye(2, dtype=jnp.float32), A)


def _dup(v):
    return jnp.concatenate([v, v]).reshape(1, -1)


def kernel(x, edge_index,
           W1_0, b1_0, W2_0, b2_0, g_0, be_0,
           W1_1, b1_1, W2_1, b2_1, g_1, be_1,
           W1_2, b1_2, W2_2, b2_2, g_2, be_2,
           W1_3, b1_3, W2_3, b2_3):
    src = edge_index[0].astype(jnp.int32)
    dst = edge_index[1].astype(jnp.int32)
    src2 = jnp.pad(((src % HALF) * 2 + src // HALF).reshape(NW, EPT),
                   ((0, 0), (0, EPT_P - EPT)))
    dst2 = jnp.pad(((dst % HALF) * 2 + dst // HALF).reshape(NW, EPT),
                   ((0, 0), (0, EPT_P - EPT)), constant_values=N)
    src3 = src2.reshape(NW, NCHUNKS, CHUNK)
    dst3 = dst2.reshape(NW, NCHUNKS, CHUNK)
    zeros_z = jnp.zeros((ZR, H), jnp.float32)
    Mavg = _bd(jnp.full((H, H), 1.0 / H, jnp.float32))

    p = _tc_project0(x, _bd(W1_0))
    for b1, W2, b2, g, be, W1n in (
            (b1_0, W2_0, b2_0, g_0, be_0, W1_1),
            (b1_1, W2_1, b2_1, g_1, be_1, W1_2),
            (b1_2, W2_2, b2_2, g_2, be_2, W1_3)):
        acc2 = _sc_segment_sum(p.reshape(N, H), src3, dst3, zeros_z)
        p = _tc_block(p, acc2.reshape(NC, NPAD // 2, HP), _dup(b1), _bd(W2),
                      _dup(b2), Mavg, _dup(g), _dup(be), _bd(W1n))
    acc2 = _sc_segment_sum(p.reshape(N, H), src3, dst3, zeros_z)
    out2 = _tc_final(p, acc2.reshape(NC, NPAD // 2, HP), _dup(b1_3),
                     _bd(W2_3), _dup(b2_3))
    return out2.reshape(N, DOUT)


# speedup vs baseline: 1.0209x; 1.0209x over previous
import functools

import jax
import jax.numpy as jnp
from jax import lax
from jax.experimental import pallas as pl
from jax.experimental.pallas import tpu as pltpu
from jax.experimental.pallas import tpu_sc as plsc

N, E, DIN, H, DOUT = 10000, 320000, 128, 64, 128

NC, NS = 2, 16
NW = NC * NS
EPT = E // NW
CHUNK = 128
EPT_P = 10240
NCHUNKS = EPT_P // CHUNK
NBUF = 4
NITER = NCHUNKS // NBUF
NPAD = 10240
ROWS_PT = NPAD // NS
ZR = 128

_PREC = jax.lax.Precision.HIGHEST


def _dot(a, b):
    return jax.lax.dot_general(a, b, (((1,), (0,)), ((), ())),
                               precision=_PREC,
                               preferred_element_type=jnp.float32)


def _sc_segment_sum(p, src3, dst3, zeros_z):
    mesh = plsc.VectorSubcoreMesh(core_axis_name="c", subcore_axis_name="s")

    @functools.partial(
        pl.kernel,
        mesh=mesh,
        compiler_params=pltpu.CompilerParams(use_tc_tiling_on_sc=False),
        out_type=jax.ShapeDtypeStruct((NC, NPAD, H), jnp.float32),
        scratch_types=[
            pltpu.VMEM((NCHUNKS, CHUNK), jnp.int32),
            pltpu.VMEM((NCHUNKS, CHUNK), jnp.int32),
            pltpu.VMEM((CHUNK, H), jnp.float32),
            pltpu.VMEM((CHUNK, H), jnp.float32),
            pltpu.VMEM((CHUNK, H), jnp.float32),
            pltpu.VMEM((CHUNK, H), jnp.float32),
            pltpu.VMEM((ZR, H), jnp.float32),
            pltpu.VMEM_SHARED((NPAD, H), jnp.float32),
            pltpu.SemaphoreType.DMA,
            pltpu.SemaphoreType.DMA,
            pltpu.SemaphoreType.DMA,
            pltpu.SemaphoreType.DMA,
        ],
    )
    def k(p_hbm, src_hbm, dst_hbm, z_hbm, out_hbm, sbuf, dbuf,
          r0, r1, r2, r3, zbuf, acc, g0, g1, g2, g3):
        rows = [r0, r1, r2, r3]
        gsem = [g0, g1, g2, g3]
        c = lax.axis_index("c")
        s = lax.axis_index("s")
        w = c * NS + s

        pltpu.sync_copy(src_hbm.at[w], sbuf)
        pltpu.sync_copy(dst_hbm.at[w], dbuf)
        for b in range(NBUF):
            pltpu.async_copy(p_hbm.at[sbuf.at[b]], rows[b], gsem[b])

        pltpu.sync_copy(z_hbm, zbuf)
        for j in range(ROWS_PT // ZR):
            pltpu.sync_copy(zbuf, acc.at[pl.ds(s * ROWS_PT + j * ZR, ZR)])
        plsc.subcore_barrier()

        def body(i, _):
            for b in range(NBUF):
                j = i * NBUF + b
                pltpu.make_async_copy(
                    p_hbm.at[sbuf.at[j]], rows[b], gsem[b]).wait()
                pltpu.sync_copy(rows[b], acc.at[dbuf.at[j]], add=True)

                @pl.when(i < NITER - 1)
                def _():
                    pltpu.async_copy(
                        p_hbm.at[sbuf.at[j + NBUF]], rows[b], gsem[b])
            return 0

        lax.fori_loop(0, NITER, body, 0)
        plsc.subcore_barrier()

        for j in range(ROWS_PT // ZR):
            r0_ = s * ROWS_PT + j * ZR
            pltpu.async_copy(acc.at[pl.ds(r0_, ZR)],
                             out_hbm.at[c].at[pl.ds(r0_, ZR)], gsem[j % NBUF])
        for j in range(ROWS_PT // ZR):
            r0_ = s * ROWS_PT + j * ZR
            pltpu.make_async_copy(acc.at[pl.ds(r0_, ZR)],
                                  out_hbm.at[c].at[pl.ds(r0_, ZR)],
                                  gsem[j % NBUF]).wait()

    return k(p, src3, dst3, zeros_z)


_BR = 2000


def _tc_project0(x, W1):
    def body(x_ref, w_ref, o_ref):
        o_ref[...] = _dot(x_ref[...], w_ref[...])

    return pl.pallas_call(
        body,
        grid=(N // _BR,),
        in_specs=[
            pl.BlockSpec((_BR, DIN), lambda i: (i, 0)),
            pl.BlockSpec((DIN, H), lambda i: (0, 0)),
        ],
        out_specs=pl.BlockSpec((_BR, H), lambda i: (i, 0)),
        out_shape=jax.ShapeDtypeStruct((N, H), jnp.float32),
    )(x, W1)


def _tc_block(p, acc2, b1, W2, b2, g, be, W1n):
    def body(p_ref, a_ref, b1_ref, w2_ref, b2_ref, g_ref, be_ref, w1n_ref,
             o_ref):
        t = p_ref[...] + a_ref[0] + a_ref[1] + b1_ref[...]
        t = jnp.maximum(t, 0.0)
        u = _dot(t, w2_ref[...]) + b2_ref[...]
        mu = jnp.mean(u, axis=-1, keepdims=True)
        var = jnp.mean((u - mu) ** 2, axis=-1, keepdims=True)
        v = (u - mu) / jnp.sqrt(var + 1e-5) * g_ref[...] + be_ref[...]
        v = jnp.maximum(v, 0.0)
        o_ref[...] = _dot(v, w1n_ref[...])

    return pl.pallas_call(
        body,
        grid=(N // _BR,),
        in_specs=[
            pl.BlockSpec((_BR, H), lambda i: (i, 0)),
            pl.BlockSpec((NC, _BR, H), lambda i: (0, i, 0)),
            pl.BlockSpec((1, H), lambda i: (0, 0)),
            pl.BlockSpec((H, H), lambda i: (0, 0)),
            pl.BlockSpec((1, H), lambda i: (0, 0)),
            pl.BlockSpec((1, H), lambda i: (0, 0)),
            pl.BlockSpec((1, H), lambda i: (0, 0)),
            pl.BlockSpec((H, H), lambda i: (0, 0)),
        ],
        out_specs=pl.BlockSpec((_BR, H), lambda i: (i, 0)),
        out_shape=jax.ShapeDtypeStruct((N, H), jnp.float32),
    )(p, acc2, b1.reshape(1, H), W2, b2.reshape(1, H), g.reshape(1, H),
      be.reshape(1, H), W1n)


def _tc_final(p, acc2, b1, W2, b2):
    def body(p_ref, a_ref, b1_ref, w2_ref, b2_ref, o_ref):
        t = p_ref[...] + a_ref[0] + a_ref[1] + b1_ref[...]
        t = jnp.maximum(t, 0.0)
        o_ref[...] = _dot(t, w2_ref[...]) + b2_ref[...]

    return pl.pallas_call(
        body,
        grid=(N // _BR,),
        in_specs=[
            pl.BlockSpec((_BR, H), lambda i: (i, 0)),
            pl.BlockSpec((NC, _BR, H), lambda i: (0, i, 0)),
            pl.BlockSpec((1, H), lambda i: (0, 0)),
            pl.BlockSpec((H, DOUT), lambda i: (0, 0)),
            pl.BlockSpec((1, DOUT), lambda i: (0, 0)),
        ],
        out_specs=pl.BlockSpec((_BR, DOUT), lambda i: (i, 0)),
        out_shape=jax.ShapeDtypeStruct((N, DOUT), jnp.float32),
    )(p, acc2, b1.reshape(1, H), W2, b2.reshape(1, DOUT))


def kernel(x, edge_index,
           W1_0, b1_0, W2_0, b2_0, g_0, be_0,
           W1_1, b1_1, W2_1, b2_1, g_1, be_1,
           W1_2, b1_2, W2_2, b2_2, g_2, be_2,
           W1_3, b1_3, W2_3, b2_3):
    src2 = jnp.pad(edge_index[0].astype(jnp.int32).reshape(NW, EPT),
                   ((0, 0), (0, EPT_P - EPT)))
    dst2 = jnp.pad(edge_index[1].astype(jnp.int32).reshape(NW, EPT),
                   ((0, 0), (0, EPT_P - EPT)), constant_values=N)
    src3 = src2.reshape(NW, NCHUNKS, CHUNK)
    dst3 = dst2.reshape(NW, NCHUNKS, CHUNK)
    zeros_z = jnp.zeros((ZR, H), jnp.float32)

    p = _tc_project0(x, W1_0)
    acc2 = _sc_segment_sum(p, src3, dst3, zeros_z)
    p = _tc_block(p, acc2, b1_0, W2_0, b2_0, g_0, be_0, W1_1)
    acc2 = _sc_segment_sum(p, src3, dst3, zeros_z)
    p = _tc_block(p, acc2, b1_1, W2_1, b2_1, g_1, be_1, W1_2)
    acc2 = _sc_segment_sum(p, src3, dst3, zeros_z)
    p = _tc_block(p, acc2, b1_2, W2_2, b2_2, g_2, be_2, W1_3)
    acc2 = _sc_segment_sum(p, src3, dst3, zeros_z)
    return _tc_final(p, acc2, b1_3, W2_3, b2_3)


# speedup vs baseline: 2.5442x; 2.4922x over previous
import functools

import jax
import jax.numpy as jnp
from jax import lax
from jax.experimental import pallas as pl
from jax.experimental.pallas import tpu as pltpu
from jax.experimental.pallas import tpu_sc as plsc

N, E, DIN, H, DOUT = 10000, 320000, 128, 64, 128

NC, NS = 2, 16
NW = NC * NS
EPT = E // NW
CHUNK = 100
EPT_P = EPT
NCHUNKS = EPT_P // CHUNK
NBUF = 4
NITER = NCHUNKS // NBUF
NPAD = 10240
ROWS_PT = NPAD // NS
ZR = 128

_PREC = jax.lax.Precision.HIGHEST


def _dot(a, b):
    return jax.lax.dot_general(a, b, (((1,), (0,)), ((), ())),
                               precision=_PREC,
                               preferred_element_type=jnp.float32)


def _sc_segment_sum(p, src3, dst3, zeros_z):
    mesh = plsc.VectorSubcoreMesh(core_axis_name="c", subcore_axis_name="s")

    @functools.partial(
        pl.kernel,
        mesh=mesh,
        compiler_params=pltpu.CompilerParams(use_tc_tiling_on_sc=False),
        out_type=jax.ShapeDtypeStruct((NC, NPAD, H), jnp.float32),
        scratch_types=[
            pltpu.VMEM((NCHUNKS, CHUNK), jnp.int32),
            pltpu.VMEM((NCHUNKS, CHUNK), jnp.int32),
            pltpu.VMEM((CHUNK, H), jnp.float32),
            pltpu.VMEM((CHUNK, H), jnp.float32),
            pltpu.VMEM((CHUNK, H), jnp.float32),
            pltpu.VMEM((CHUNK, H), jnp.float32),
            pltpu.VMEM((ZR, H), jnp.float32),
            pltpu.VMEM_SHARED((NPAD, H), jnp.float32),
            pltpu.SemaphoreType.DMA,
            pltpu.SemaphoreType.DMA,
            pltpu.SemaphoreType.DMA,
            pltpu.SemaphoreType.DMA,
        ],
    )
    def k(p_hbm, src_hbm, dst_hbm, z_hbm, out_hbm, sbuf, dbuf,
          r0, r1, r2, r3, zbuf, acc, g0, g1, g2, g3):
        rows = [r0, r1, r2, r3]
        gsem = [g0, g1, g2, g3]
        c = lax.axis_index("c")
        s = lax.axis_index("s")
        w = c * NS + s

        pltpu.sync_copy(src_hbm.at[w], sbuf)
        pltpu.sync_copy(dst_hbm.at[w], dbuf)
        for b in range(NBUF):
            pltpu.async_copy(p_hbm.at[sbuf.at[b]], rows[b], gsem[b])

        pltpu.sync_copy(z_hbm, zbuf)
        for j in range(ROWS_PT // ZR):
            pltpu.sync_copy(zbuf, acc.at[pl.ds(s * ROWS_PT + j * ZR, ZR)])
        plsc.subcore_barrier()

        def body(i, _):
            for b in range(NBUF):
                j = i * NBUF + b
                pltpu.make_async_copy(
                    p_hbm.at[sbuf.at[j]], rows[b], gsem[b]).wait()
                pltpu.sync_copy(rows[b], acc.at[dbuf.at[j]], add=True)

                @pl.when(i < NITER - 1)
                def _():
                    pltpu.async_copy(
                        p_hbm.at[sbuf.at[j + NBUF]], rows[b], gsem[b])
            return 0

        lax.fori_loop(0, NITER, body, 0)
        plsc.subcore_barrier()

        for j in range(ROWS_PT // ZR):
            r0_ = s * ROWS_PT + j * ZR
            pltpu.async_copy(acc.at[pl.ds(r0_, ZR)],
                             out_hbm.at[c].at[pl.ds(r0_, ZR)], gsem[j % NBUF])
        for j in range(ROWS_PT // ZR):
            r0_ = s * ROWS_PT + j * ZR
            pltpu.make_async_copy(acc.at[pl.ds(r0_, ZR)],
                                  out_hbm.at[c].at[pl.ds(r0_, ZR)],
                                  gsem[j % NBUF]).wait()

    return k(p, src3, dst3, zeros_z)


_BR = 2000


def _tc_project0(x, W1):
    def body(x_ref, w_ref, o_ref):
        o_ref[...] = _dot(x_ref[...], w_ref[...])

    return pl.pallas_call(
        body,
        grid=(N // _BR,),
        in_specs=[
            pl.BlockSpec((_BR, DIN), lambda i: (i, 0)),
            pl.BlockSpec((DIN, H), lambda i: (0, 0)),
        ],
        out_specs=pl.BlockSpec((_BR, H), lambda i: (i, 0)),
        out_shape=jax.ShapeDtypeStruct((N, H), jnp.float32),
    )(x, W1)


def _tc_block(p, acc2, b1, W2, b2, g, be, W1n):
    def body(p_ref, a_ref, b1_ref, w2_ref, b2_ref, g_ref, be_ref, w1n_ref,
             o_ref):
        t = p_ref[...] + a_ref[0] + a_ref[1] + b1_ref[...]
        t = jnp.maximum(t, 0.0)
        u = _dot(t, w2_ref[...]) + b2_ref[...]
        mu = jnp.mean(u, axis=-1, keepdims=True)
        var = jnp.mean((u - mu) ** 2, axis=-1, keepdims=True)
        v = (u - mu) / jnp.sqrt(var + 1e-5) * g_ref[...] + be_ref[...]
        v = jnp.maximum(v, 0.0)
        o_ref[...] = _dot(v, w1n_ref[...])

    return pl.pallas_call(
        body,
        grid=(N // _BR,),
        in_specs=[
            pl.BlockSpec((_BR, H), lambda i: (i, 0)),
            pl.BlockSpec((NC, _BR, H), lambda i: (0, i, 0)),
            pl.BlockSpec((1, H), lambda i: (0, 0)),
            pl.BlockSpec((H, H), lambda i: (0, 0)),
            pl.BlockSpec((1, H), lambda i: (0, 0)),
            pl.BlockSpec((1, H), lambda i: (0, 0)),
            pl.BlockSpec((1, H), lambda i: (0, 0)),
            pl.BlockSpec((H, H), lambda i: (0, 0)),
        ],
        out_specs=pl.BlockSpec((_BR, H), lambda i: (i, 0)),
        out_shape=jax.ShapeDtypeStruct((N, H), jnp.float32),
    )(p, acc2, b1.reshape(1, H), W2, b2.reshape(1, H), g.reshape(1, H),
      be.reshape(1, H), W1n)


def _tc_final(p, acc2, b1, W2, b2):
    def body(p_ref, a_ref, b1_ref, w2_ref, b2_ref, o_ref):
        t = p_ref[...] + a_ref[0] + a_ref[1] + b1_ref[...]
        t = jnp.maximum(t, 0.0)
        o_ref[...] = _dot(t, w2_ref[...]) + b2_ref[...]

    return pl.pallas_call(
        body,
        grid=(N // _BR,),
        in_specs=[
            pl.BlockSpec((_BR, H), lambda i: (i, 0)),
            pl.BlockSpec((NC, _BR, H), lambda i: (0, i, 0)),
            pl.BlockSpec((1, H), lambda i: (0, 0)),
            pl.BlockSpec((H, DOUT), lambda i: (0, 0)),
            pl.BlockSpec((1, DOUT), lambda i: (0, 0)),
        ],
        out_specs=pl.BlockSpec((_BR, DOUT), lambda i: (i, 0)),
        out_shape=jax.ShapeDtypeStruct((N, DOUT), jnp.float32),
    )(p, acc2, b1.reshape(1, H), W2, b2.reshape(1, DOUT))


def kernel(x, edge_index,
           W1_0, b1_0, W2_0, b2_0, g_0, be_0,
           W1_1, b1_1, W2_1, b2_1, g_1, be_1,
           W1_2, b1_2, W2_2, b2_2, g_2, be_2,
           W1_3, b1_3, W2_3, b2_3):
    src3 = edge_index[0].astype(jnp.int32).reshape(NW, NCHUNKS, CHUNK)
    dst3 = edge_index[1].astype(jnp.int32).reshape(NW, NCHUNKS, CHUNK)
    zeros_z = jnp.zeros((ZR, H), jnp.float32)

    p = _tc_project0(x, W1_0)
    acc2 = _sc_segment_sum(p, src3, dst3, zeros_z)
    p = _tc_block(p, acc2, b1_0, W2_0, b2_0, g_0, be_0, W1_1)
    acc2 = _sc_segment_sum(p, src3, dst3, zeros_z)
    p = _tc_block(p, acc2, b1_1, W2_1, b2_1, g_1, be_1, W1_2)
    acc2 = _sc_segment_sum(p, src3, dst3, zeros_z)
    p = _tc_block(p, acc2, b1_2, W2_2, b2_2, g_2, be_2, W1_3)
    acc2 = _sc_segment_sum(p, src3, dst3, zeros_z)
    return _tc_final(p, acc2, b1_3, W2_3, b2_3)
